# Initial kernel scaffold; baseline (speedup 1.0000x reference)
import jax
import jax.numpy as jnp
from jax.experimental import pallas as pl


def kernel(x, new_weight, orig_weight):
    raise NotImplementedError("write your pallas kernel here")


# speedup vs baseline: 1.0312x; 1.0312x over previous
import jax
import jax.numpy as jnp
from jax.experimental import pallas as pl
from jax.experimental.pallas import tpu as pltpu

_BN = 512


def _acke_body(x_ref, nw_ref, ow_ref, o1_ref, o2_ref):
    x = x_ref[...]
    dims = (((1,), (1,)), ((), ()))
    o1_ref[...] = jax.lax.dot_general(
        x, nw_ref[...], dims, preferred_element_type=jnp.float32)
    o2_ref[...] = jax.lax.dot_general(
        x, ow_ref[...], dims, preferred_element_type=jnp.float32)


@jax.jit
def kernel(x, new_weight, orig_weight):
    b, k = x.shape
    n = new_weight.shape[0]
    grid = (n // _BN,)
    out_shape = jax.ShapeDtypeStruct((b, n), jnp.float32)
    call = pl.pallas_call(
        _acke_body,
        grid=grid,
        in_specs=[
            pl.BlockSpec((b, k), lambda j: (0, 0)),
            pl.BlockSpec((_BN, k), lambda j: (j, 0)),
            pl.BlockSpec((_BN, k), lambda j: (j, 0)),
        ],
        out_specs=[
            pl.BlockSpec((b, _BN), lambda j: (0, j)),
            pl.BlockSpec((b, _BN), lambda j: (0, j)),
        ],
        out_shape=[out_shape, out_shape],
        compiler_params=pltpu.CompilerParams(
            dimension_semantics=("arbitrary",)),
    )
    layer_out, original_layer_output = call(x, new_weight, orig_weight)
    return (layer_out, original_layer_output)
